# baseline (speedup 1.0000x reference)
import functools

import jax
import jax.numpy as jnp
from jax import lax
from jax.experimental import pallas as pl
from jax.experimental.pallas import tpu as pltpu
from jax.experimental.pallas import tpu_sc as plsc

N = 10000
NP = 10240
E = 320000
D_IN = 128
D = 256
D2 = 512
L = 12
EPS = 1e-7

NW = 32
NODES_PER_W = NP // NW
GROUP = 64
NGRP = NODES_PER_W // GROUP
K = 32
SE = 4096
RP_LEN = (NW - 1) * NODES_PER_W + 336


def _sc_segment_sums(z, src_s, dst_s, rp):
    mesh = plsc.VectorSubcoreMesh(core_axis_name="c", subcore_axis_name="s")

    @functools.partial(
        pl.kernel,
        out_type=jax.ShapeDtypeStruct((NP, D2), jnp.float32),
        mesh=mesh,
        scratch_types=[
            pltpu.VMEM((SE,), jnp.int32),
            pltpu.VMEM((SE,), jnp.int32),
            pltpu.VMEM((2, K, D2), jnp.float32),
            pltpu.VMEM((GROUP + 1, D2), jnp.float32),
            pltpu.VMEM((336,), jnp.int32),
            pltpu.SemaphoreType.DMA,
            pltpu.SemaphoreType.DMA,
        ],
    )
    def k(z_hbm, src_hbm, dst_hbm, rp_hbm, out_hbm,
          idx_v, dst_v, rows_v, acc_v, rp_v, sem0, sem1):
        cid = lax.axis_index("c")
        sid = lax.axis_index("s")
        wid = sid * 2 + cid
        node0 = wid * NODES_PER_W
        pltpu.sync_copy(rp_hbm.at[pl.ds(node0, 336)], rp_v)

        zero16 = jnp.zeros((16,), jnp.float32)
        iota16 = lax.iota(jnp.int32, 16)
        sems = (sem0, sem1)

        def gather(ch, b):
            pltpu.async_copy(
                z_hbm.at[idx_v.at[pl.ds(ch * K, K)]], rows_v.at[b], sems[b])

        def gb(grp, gcarry):
            gl = grp * GROUP
            gbase = node0 + gl

            def zrow(r, carry):
                for cc in range(D2 // 16):
                    acc_v[r, pl.ds(cc * 16, 16)] = zero16
                return carry
            lax.fori_loop(0, GROUP + 1, zrow, 0)

            e_lo = rp_v[pl.ds(gl, 16)][0]
            e_hi = rp_v[pl.ds(gl + GROUP, 16)][0]
            sc0 = (e_lo // K) * K
            nsc = (e_hi - sc0 + SE - 1) // SE

            def sb(s, scarry):
                sbase = sc0 + s * SE
                pltpu.sync_copy(src_hbm.at[pl.ds(sbase, SE)], idx_v)
                pltpu.sync_copy(dst_hbm.at[pl.ds(sbase, SE)], dst_v)
                lim = jnp.minimum(sbase + SE, e_hi)
                nch = (lim - sbase + K - 1) // K

                @pl.when(nch > 0)
                def _():
                    gather(0, 0)

                def pair(g, carry):
                    for b in range(2):
                        ch = g * 2 + b

                        @pl.when(ch < nch)
                        def _():
                            @pl.when(ch + 1 < nch)
                            def _():
                                gather(ch + 1, 1 - b)
                            pltpu.make_async_copy(
                                z_hbm.at[idx_v.at[pl.ds(ch * K, K)]],
                                rows_v.at[b], sems[b]).wait()
                            for jj in range(K // 16):
                                for j16 in range(16):
                                    j = jj * 16 + j16
                                    for cc in range(D2 // 16):
                                        sl = pl.ds(cc * 16, 16)
                                        plsc.addupdate(acc_v.at[j, sl],
                                                       rows_v[b, j, sl])
                    return carry
                lax.fori_loop(0, (nch + 1) // 2, pair, 0)
                return scarry
            lax.fori_loop(0, nsc, sb, 0)

            pltpu.sync_copy(acc_v.at[:GROUP],
                            out_hbm.at[pl.ds(gbase, GROUP), :])
            return gcarry
        lax.fori_loop(0, NGRP, gb, 0)

    return k(z, src_s, dst_s, rp)


BM = 256
GRID = NP // BM


def _write_z(z_ref, g, tval):
    q = jnp.exp(g * tval)
    z_ref[:, :D] = q
    z_ref[:, D:] = q * g


def _enc_body(t_ref, x_ref, w_ref, b_ref, h_ref, z_ref):
    h = jnp.dot(x_ref[...], w_ref[...],
                preferred_element_type=jnp.float32) + b_ref[...]
    h_ref[...] = h
    g = jnp.maximum(h, 0.0) + EPS
    _write_z(z_ref, g, t_ref[0, 0])


def _layer_norm(o, w, b):
    mu = jnp.mean(o, axis=-1, keepdims=True)
    var = jnp.mean((o - mu) ** 2, axis=-1, keepdims=True)
    return (o - mu) / jnp.sqrt(var + 1e-5) * w + b


def _mlp(hin, dn, W1_ref, b1_ref, lw_ref, lb_ref, W2_ref, b2_ref):
    den = dn[:, :D]
    num = dn[:, D:]
    o = hin + num / (den + 1e-16)
    o = jnp.dot(o, W1_ref[...], preferred_element_type=jnp.float32) + b1_ref[...]
    o = _layer_norm(o, lw_ref[...], lb_ref[...])
    o = jnp.maximum(o, 0.0)
    o = jnp.dot(o, W2_ref[...], preferred_element_type=jnp.float32) + b2_ref[...]
    return o


def _layer_body(res_scale, tn_ref, h_ref, hin_ref, dn_ref,
                W1_ref, b1_ref, lw_ref, lb_ref, W2_ref, b2_ref,
                nw_ref, nb_ref, hn_ref, hin2_ref, z_ref):
    o = _mlp(hin_ref[...], dn_ref[...], W1_ref, b1_ref, lw_ref, lb_ref,
             W2_ref, b2_ref)
    hn = res_scale * h_ref[...] + o
    hn_ref[...] = hn
    hin2 = jnp.maximum(_layer_norm(hn, nw_ref[...], nb_ref[...]), 0.0)
    hin2_ref[...] = hin2
    _write_z(z_ref, hin2 + EPS, tn_ref[0, 0])


def _final_body(h_ref, hin_ref, dn_ref,
                W1_ref, b1_ref, lw_ref, lb_ref, W2_ref, b2_ref,
                nw_ref, nb_ref, out_ref):
    o = _mlp(hin_ref[...], dn_ref[...], W1_ref, b1_ref, lw_ref, lb_ref,
             W2_ref, b2_ref)
    hn = h_ref[...] + o
    out_ref[...] = jnp.maximum(_layer_norm(hn, nw_ref[...], nb_ref[...]), 0.0)


def _row_spec(cols):
    return pl.BlockSpec((BM, cols), lambda i: (i, 0))


def _full_spec(shape):
    nd = len(shape)
    return pl.BlockSpec(shape, lambda i: (0,) * nd)


def _smem_spec():
    return pl.BlockSpec((1, 1), lambda i: (0, 0), memory_space=pltpu.SMEM)


def _enc_call(t0, x_p, enc_W, enc_b):
    return pl.pallas_call(
        _enc_body,
        grid=(GRID,),
        in_specs=[_smem_spec(), _row_spec(D_IN), _full_spec((D_IN, D)),
                  _full_spec((1, D))],
        out_specs=[_row_spec(D), _row_spec(D2)],
        out_shape=[jax.ShapeDtypeStruct((NP, D), jnp.float32),
                   jax.ShapeDtypeStruct((NP, D2), jnp.float32)],
    )(t0, x_p, enc_W, enc_b)


def _layer_call(res_scale, tn, h, hin, dn, W1, b1, lw, lb, W2, b2, nw, nb):
    return pl.pallas_call(
        functools.partial(_layer_body, res_scale),
        grid=(GRID,),
        in_specs=[_smem_spec(), _row_spec(D), _row_spec(D), _row_spec(D2),
                  _full_spec((D, D2)), _full_spec((1, D2)),
                  _full_spec((1, D2)), _full_spec((1, D2)),
                  _full_spec((D2, D)), _full_spec((1, D)),
                  _full_spec((1, D)), _full_spec((1, D))],
        out_specs=[_row_spec(D), _row_spec(D), _row_spec(D2)],
        out_shape=[jax.ShapeDtypeStruct((NP, D), jnp.float32),
                   jax.ShapeDtypeStruct((NP, D), jnp.float32),
                   jax.ShapeDtypeStruct((NP, D2), jnp.float32)],
    )(tn, h, hin, dn, W1, b1, lw, lb, W2, b2, nw, nb)


def _final_call(h, hin, dn, W1, b1, lw, lb, W2, b2, nw, nb):
    return pl.pallas_call(
        _final_body,
        grid=(GRID,),
        in_specs=[_row_spec(D), _row_spec(D), _row_spec(D2),
                  _full_spec((D, D2)), _full_spec((1, D2)),
                  _full_spec((1, D2)), _full_spec((1, D2)),
                  _full_spec((D2, D)), _full_spec((1, D)),
                  _full_spec((1, D)), _full_spec((1, D))],
        out_specs=_row_spec(D),
        out_shape=jax.ShapeDtypeStruct((NP, D), jnp.float32),
    )(h, hin, dn, W1, b1, lw, lb, W2, b2, nw, nb)


def kernel(x, edge_index, enc_W, enc_b, W1, b1, ln1_w, ln1_b, W2, b2, t,
           norm_w, norm_b):
    src = edge_index[0].astype(jnp.int32)
    dst = edge_index[1].astype(jnp.int32)

    order = jnp.argsort(dst)
    src_s = jnp.pad(src[order], (0, SE))
    dst_s = jnp.pad(dst[order], (0, SE))
    counts = jnp.bincount(dst, length=NP)
    rp = jnp.concatenate([jnp.zeros((1,), jnp.int32),
                          jnp.cumsum(counts).astype(jnp.int32)])
    rp = jnp.pad(rp, (0, RP_LEN - (NP + 1)), constant_values=E)

    x_p = jnp.pad(x, ((0, NP - N), (0, 0)))
    b1r = b1.reshape(L, 1, D2)
    b2r = b2.reshape(L, 1, D)
    lwr = ln1_w.reshape(L, 1, D2)
    lbr = ln1_b.reshape(L, 1, D2)
    nwr = norm_w.reshape(L, 1, D)
    nbr = norm_b.reshape(L, 1, D)
    tr = t.reshape(L, 1, 1)

    h, z = _enc_call(tr[0], x_p, enc_W, enc_b.reshape(1, D))
    hin = h
    for i in range(L - 1):
        dn = _sc_segment_sums(z, src_s, dst_s, rp)
        res_scale = 0.0 if i == 0 else 1.0
        h, hin, z = _layer_call(res_scale, tr[i + 1], h, hin, dn,
                                W1[i], b1r[i], lwr[i], lbr[i], W2[i], b2r[i],
                                nwr[i + 1], nbr[i + 1])
    dn = _sc_segment_sums(z, src_s, dst_s, rp)
    out = _final_call(h, hin, dn, W1[L - 1], b1r[L - 1], lwr[L - 1],
                      lbr[L - 1], W2[L - 1], b2r[L - 1], nwr[0], nbr[0])
    return out[:N]


# speedup vs baseline: 4.8517x; 4.8517x over previous
import functools

import jax
import jax.numpy as jnp
from jax import lax
from jax.experimental import pallas as pl
from jax.experimental.pallas import tpu as pltpu
from jax.experimental.pallas import tpu_sc as plsc

N = 10000
NP = 10240
E = 320000
D_IN = 128
D = 256
D2 = 512
L = 12
EPS = 1e-7

NW = 32
NODES_PER_W = NP // NW
GROUP = 64
NGRP = NODES_PER_W // GROUP
K = 32
SE = 4096
RP_LEN = (NW - 1) * NODES_PER_W + 336


def _sc_segment_sums(z, src_s, dst_s, rp):
    mesh = plsc.VectorSubcoreMesh(core_axis_name="c", subcore_axis_name="s")

    @functools.partial(
        pl.kernel,
        out_type=jax.ShapeDtypeStruct((NP, D2), jnp.float32),
        mesh=mesh,
        scratch_types=[
            pltpu.VMEM((SE,), jnp.int32),
            pltpu.VMEM((SE,), jnp.int32),
            pltpu.VMEM((2, K, D2), jnp.float32),
            pltpu.VMEM((GROUP + 1, D2), jnp.float32),
            pltpu.VMEM((336,), jnp.int32),
            pltpu.SemaphoreType.DMA,
            pltpu.SemaphoreType.DMA,
        ],
    )
    def k(z_hbm, src_hbm, dst_hbm, rp_hbm, out_hbm,
          idx_v, dst_v, rows_v, acc_v, rp_v, sem0, sem1):
        cid = lax.axis_index("c")
        sid = lax.axis_index("s")
        wid = sid * 2 + cid
        node0 = wid * NODES_PER_W
        pltpu.sync_copy(rp_hbm.at[pl.ds(node0, 336)], rp_v)

        zero16 = jnp.zeros((16,), jnp.float32)
        iota16 = lax.iota(jnp.int32, 16)
        sems = (sem0, sem1)

        def gather(ch, b):
            pltpu.async_copy(
                z_hbm.at[idx_v.at[pl.ds(ch * K, K)]], rows_v.at[b], sems[b])

        def gb(grp, gcarry):
            gl = grp * GROUP
            gbase = node0 + gl

            def zrow(r, carry):
                for cc in range(D2 // 16):
                    acc_v[r, pl.ds(cc * 16, 16)] = zero16
                return carry
            lax.fori_loop(0, GROUP + 1, zrow, 0)

            e_lo = rp_v[pl.ds(gl, 16)][0]
            e_hi = rp_v[pl.ds(gl + GROUP, 16)][0]
            sc0 = (e_lo // K) * K
            nsc = (e_hi - sc0 + SE - 1) // SE

            def sb(s, scarry):
                sbase = sc0 + s * SE
                pltpu.sync_copy(src_hbm.at[pl.ds(sbase, SE)], idx_v)
                pltpu.sync_copy(dst_hbm.at[pl.ds(sbase, SE)], dst_v)
                lim = jnp.minimum(sbase + SE, e_hi)
                nch = (lim - sbase + K - 1) // K

                @pl.when(nch > 0)
                def _():
                    gather(0, 0)

                def pair(g, carry):
                    for b in range(2):
                        ch = g * 2 + b

                        @pl.when(ch < nch)
                        def _():
                            @pl.when(ch + 1 < nch)
                            def _():
                                gather(ch + 1, 1 - b)
                            pltpu.make_async_copy(
                                z_hbm.at[idx_v.at[pl.ds(ch * K, K)]],
                                rows_v.at[b], sems[b]).wait()
                            for jj in range(K // 16):
                                off = ch * K + jj * 16
                                dv = dst_v[pl.ds(off, 16)] - gbase
                                ev = sbase + off + iota16
                                valid = jnp.logical_and(ev >= e_lo,
                                                        ev < e_hi)
                                dlv = jnp.where(valid, dv, GROUP)
                                for j16 in range(16):
                                    j = jj * 16 + j16
                                    dl = dlv[j16]

                                    @functools.partial(
                                        plsc.parallel_loop, 0, D2 // 16,
                                        unroll=8)
                                    def _(cc):
                                        sl = pl.ds(cc * 16, 16)
                                        plsc.addupdate(acc_v.at[dl, sl],
                                                       rows_v[b, j, sl])
                    return carry
                lax.fori_loop(0, (nch + 1) // 2, pair, 0)
                return scarry
            lax.fori_loop(0, nsc, sb, 0)

            pltpu.sync_copy(acc_v.at[:GROUP],
                            out_hbm.at[pl.ds(gbase, GROUP), :])
            return gcarry
        lax.fori_loop(0, NGRP, gb, 0)

    return k(z, src_s, dst_s, rp)


BM = 256
GRID = NP // BM


def _write_z(z_ref, g, tval):
    q = jnp.exp(g * tval)
    z_ref[:, :D] = q
    z_ref[:, D:] = q * g


def _enc_body(t_ref, x_ref, w_ref, b_ref, h_ref, z_ref):
    h = jnp.dot(x_ref[...], w_ref[...],
                preferred_element_type=jnp.float32) + b_ref[...]
    h_ref[...] = h
    g = jnp.maximum(h, 0.0) + EPS
    _write_z(z_ref, g, t_ref[0, 0])


def _layer_norm(o, w, b):
    mu = jnp.mean(o, axis=-1, keepdims=True)
    var = jnp.mean((o - mu) ** 2, axis=-1, keepdims=True)
    return (o - mu) / jnp.sqrt(var + 1e-5) * w + b


def _mlp(hin, dn, W1_ref, b1_ref, lw_ref, lb_ref, W2_ref, b2_ref):
    den = dn[:, :D]
    num = dn[:, D:]
    o = hin + num / (den + 1e-16)
    o = jnp.dot(o, W1_ref[...], preferred_element_type=jnp.float32) + b1_ref[...]
    o = _layer_norm(o, lw_ref[...], lb_ref[...])
    o = jnp.maximum(o, 0.0)
    o = jnp.dot(o, W2_ref[...], preferred_element_type=jnp.float32) + b2_ref[...]
    return o


def _layer_body(res_scale, tn_ref, h_ref, hin_ref, dn_ref,
                W1_ref, b1_ref, lw_ref, lb_ref, W2_ref, b2_ref,
                nw_ref, nb_ref, hn_ref, hin2_ref, z_ref):
    o = _mlp(hin_ref[...], dn_ref[...], W1_ref, b1_ref, lw_ref, lb_ref,
             W2_ref, b2_ref)
    hn = res_scale * h_ref[...] + o
    hn_ref[...] = hn
    hin2 = jnp.maximum(_layer_norm(hn, nw_ref[...], nb_ref[...]), 0.0)
    hin2_ref[...] = hin2
    _write_z(z_ref, hin2 + EPS, tn_ref[0, 0])


def _final_body(h_ref, hin_ref, dn_ref,
                W1_ref, b1_ref, lw_ref, lb_ref, W2_ref, b2_ref,
                nw_ref, nb_ref, out_ref):
    o = _mlp(hin_ref[...], dn_ref[...], W1_ref, b1_ref, lw_ref, lb_ref,
             W2_ref, b2_ref)
    hn = h_ref[...] + o
    out_ref[...] = jnp.maximum(_layer_norm(hn, nw_ref[...], nb_ref[...]), 0.0)


def _row_spec(cols):
    return pl.BlockSpec((BM, cols), lambda i: (i, 0))


def _full_spec(shape):
    nd = len(shape)
    return pl.BlockSpec(shape, lambda i: (0,) * nd)


def _smem_spec():
    return pl.BlockSpec((1, 1), lambda i: (0, 0), memory_space=pltpu.SMEM)


def _enc_call(t0, x_p, enc_W, enc_b):
    return pl.pallas_call(
        _enc_body,
        grid=(GRID,),
        in_specs=[_smem_spec(), _row_spec(D_IN), _full_spec((D_IN, D)),
                  _full_spec((1, D))],
        out_specs=[_row_spec(D), _row_spec(D2)],
        out_shape=[jax.ShapeDtypeStruct((NP, D), jnp.float32),
                   jax.ShapeDtypeStruct((NP, D2), jnp.float32)],
    )(t0, x_p, enc_W, enc_b)


def _layer_call(res_scale, tn, h, hin, dn, W1, b1, lw, lb, W2, b2, nw, nb):
    return pl.pallas_call(
        functools.partial(_layer_body, res_scale),
        grid=(GRID,),
        in_specs=[_smem_spec(), _row_spec(D), _row_spec(D), _row_spec(D2),
                  _full_spec((D, D2)), _full_spec((1, D2)),
                  _full_spec((1, D2)), _full_spec((1, D2)),
                  _full_spec((D2, D)), _full_spec((1, D)),
                  _full_spec((1, D)), _full_spec((1, D))],
        out_specs=[_row_spec(D), _row_spec(D), _row_spec(D2)],
        out_shape=[jax.ShapeDtypeStruct((NP, D), jnp.float32),
                   jax.ShapeDtypeStruct((NP, D), jnp.float32),
                   jax.ShapeDtypeStruct((NP, D2), jnp.float32)],
    )(tn, h, hin, dn, W1, b1, lw, lb, W2, b2, nw, nb)


def _final_call(h, hin, dn, W1, b1, lw, lb, W2, b2, nw, nb):
    return pl.pallas_call(
        _final_body,
        grid=(GRID,),
        in_specs=[_row_spec(D), _row_spec(D), _row_spec(D2),
                  _full_spec((D, D2)), _full_spec((1, D2)),
                  _full_spec((1, D2)), _full_spec((1, D2)),
                  _full_spec((D2, D)), _full_spec((1, D)),
                  _full_spec((1, D)), _full_spec((1, D))],
        out_specs=_row_spec(D),
        out_shape=jax.ShapeDtypeStruct((NP, D), jnp.float32),
    )(h, hin, dn, W1, b1, lw, lb, W2, b2, nw, nb)


def kernel(x, edge_index, enc_W, enc_b, W1, b1, ln1_w, ln1_b, W2, b2, t,
           norm_w, norm_b):
    src = edge_index[0].astype(jnp.int32)
    dst = edge_index[1].astype(jnp.int32)

    order = jnp.argsort(dst)
    src_s = jnp.pad(src[order], (0, SE))
    dst_s = jnp.pad(dst[order], (0, SE))
    counts = jnp.bincount(dst, length=NP)
    rp = jnp.concatenate([jnp.zeros((1,), jnp.int32),
                          jnp.cumsum(counts).astype(jnp.int32)])
    rp = jnp.pad(rp, (0, RP_LEN - (NP + 1)), constant_values=E)

    x_p = jnp.pad(x, ((0, NP - N), (0, 0)))
    b1r = b1.reshape(L, 1, D2)
    b2r = b2.reshape(L, 1, D)
    lwr = ln1_w.reshape(L, 1, D2)
    lbr = ln1_b.reshape(L, 1, D2)
    nwr = norm_w.reshape(L, 1, D)
    nbr = norm_b.reshape(L, 1, D)
    tr = t.reshape(L, 1, 1)

    h, z = _enc_call(tr[0], x_p, enc_W, enc_b.reshape(1, D))
    hin = h
    for i in range(L - 1):
        dn = _sc_segment_sums(z, src_s, dst_s, rp)
        res_scale = 0.0 if i == 0 else 1.0
        h, hin, z = _layer_call(res_scale, tr[i + 1], h, hin, dn,
                                W1[i], b1r[i], lwr[i], lbr[i], W2[i], b2r[i],
                                nwr[i + 1], nbr[i + 1])
    dn = _sc_segment_sums(z, src_s, dst_s, rp)
    out = _final_call(h, hin, dn, W1[L - 1], b1r[L - 1], lwr[L - 1],
                      lbr[L - 1], W2[L - 1], b2r[L - 1], nwr[0], nbr[0])
    return out[:N]
